# baseline (speedup 1.0000x reference)
import functools

import jax
import jax.numpy as jnp
from jax import lax
from jax.experimental import pallas as pl
from jax.experimental.pallas import tpu as pltpu
from jax.experimental.pallas import tpu_sc as plsc

B = 16384
D = 128
S = 25
L = 16
NC = 2
NS = 16
NW = NC * NS
ROWS_PER_W = B // NW
R = 4
IDX_PER_STEP = R * S
NSTEPS = ROWS_PER_W // R
RS = 2
IDX_PER_SLOT = RS * S
NSLOTS = ROWS_PER_W // RS


def _make_sc_call():
    mesh = plsc.VectorSubcoreMesh(core_axis_name="c", subcore_axis_name="s")

    @functools.partial(
        pl.kernel,
        mesh=mesh,
        out_type=jax.ShapeDtypeStruct((B, D), jnp.float32),
        scratch_types=[
            pltpu.VMEM((NSLOTS, IDX_PER_SLOT), jnp.int32),
            pltpu.VMEM((IDX_PER_STEP, D), jnp.float32),
            pltpu.VMEM((IDX_PER_STEP, D), jnp.float32),
            pltpu.VMEM((ROWS_PER_W, D), jnp.float32),
            pltpu.SemaphoreType.DMA,
            pltpu.SemaphoreType.DMA,
            pltpu.SemaphoreType.DMA,
            pltpu.SemaphoreType.DMA,
        ],
    )
    def sc_mean(table_hbm, idx_hbm, out_hbm, idx_v,
                buf0, buf1, out_v, sm0, sm1, sm2, sm3):
        wid = lax.axis_index("s") * NC + lax.axis_index("c")
        sems = (sm0, sm1, sm2, sm3)
        pltpu.sync_copy(idx_hbm.at[wid], idx_v)

        def slot_dst(h):
            buf = buf0 if h < 2 else buf1
            return buf.at[pl.ds((h % 2) * IDX_PER_SLOT, IDX_PER_SLOT)]

        def start(t, h):
            pltpu.async_copy(table_hbm.at[idx_v.at[t]], slot_dst(h), sems[h])

        def drain(t, h):
            pltpu.make_async_copy(table_hbm.at[idx_v.at[t]], slot_dst(h),
                                  sems[h]).wait()

        def compute(t, h):
            buf = buf0 if h < 2 else buf1
            row0 = (h % 2) * IDX_PER_SLOT

            def reduce_store(r, c, vals):
                while len(vals) > 1:
                    nxt = [a + b for a, b in zip(vals[0::2], vals[1::2])]
                    if len(vals) % 2:
                        nxt.append(vals[-1])
                    vals = nxt
                out_v[t * RS + r, pl.ds(c * L, L)] = vals[0] * (1.0 / S)

            prev = None
            for r in range(RS):
                for c in range(D // L):
                    vals = [buf[row0 + r * S + s, pl.ds(c * L, L)]
                            for s in range(S)]
                    if prev is not None:
                        reduce_store(*prev)
                    prev = (r, c, vals)
            reduce_store(*prev)

        start(0, 0)
        start(1, 1)
        start(2, 2)

        def step(i, carry):
            t4 = i * 4
            for j in range(4):
                t = t4 + j
                drain(t, j)

                @pl.when(t + 3 < NSLOTS)
                def _():
                    start(t + 3, (j + 3) % 4)

                compute(t, j)
            return carry

        lax.fori_loop(0, NSLOTS // 4, step, 0)
        pltpu.sync_copy(out_v, out_hbm.at[pl.ds(wid * ROWS_PER_W, ROWS_PER_W)])

    return sc_mean


_sc_mean = _make_sc_call()


def kernel(nodes, neigh_idx, num_sample, table):
    del nodes, num_sample
    idx = jnp.reshape(neigh_idx.astype(jnp.int32), (NW, NSLOTS, IDX_PER_SLOT))
    return _sc_mean(table, idx)


# speedup vs baseline: 1.0050x; 1.0050x over previous
import functools

import jax
import jax.numpy as jnp
from jax import lax
from jax.experimental import pallas as pl
from jax.experimental.pallas import tpu as pltpu
from jax.experimental.pallas import tpu_sc as plsc

B = 16384
D = 128
S = 25
L = 16
NC = 2
NS = 16
NW = NC * NS
ROWS_PER_W = B // NW
R = 4
IDX_PER_STEP = R * S
NSTEPS = ROWS_PER_W // R
RS = 2
IDX_PER_SLOT = RS * S
NSLOTS = ROWS_PER_W // RS


def _make_sc_call():
    mesh = plsc.VectorSubcoreMesh(core_axis_name="c", subcore_axis_name="s")

    @functools.partial(
        pl.kernel,
        mesh=mesh,
        out_type=jax.ShapeDtypeStruct((B, D), jnp.float32),
        scratch_types=[
            pltpu.VMEM((NSLOTS, IDX_PER_SLOT), jnp.int32),
            pltpu.VMEM((IDX_PER_STEP, D), jnp.float32),
            pltpu.VMEM((IDX_PER_STEP, D), jnp.float32),
            pltpu.VMEM((ROWS_PER_W, D), jnp.float32),
            pltpu.SemaphoreType.DMA,
            pltpu.SemaphoreType.DMA,
            pltpu.SemaphoreType.DMA,
            pltpu.SemaphoreType.DMA,
            pltpu.SemaphoreType.DMA,
        ],
    )
    def sc_mean(table_hbm, idx_hbm, out_hbm, idx_v,
                buf0, buf1, out_v, sm0, sm1, sm2, sm3, osem):
        wid = lax.axis_index("s") * NC + lax.axis_index("c")
        base_row = wid * ROWS_PER_W
        half = ROWS_PER_W // 2
        sems = (sm0, sm1, sm2, sm3)
        pltpu.sync_copy(idx_hbm.at[wid], idx_v)

        def slot_dst(h):
            buf = buf0 if h < 2 else buf1
            return buf.at[pl.ds((h % 2) * IDX_PER_SLOT, IDX_PER_SLOT)]

        def start(t, h):
            pltpu.async_copy(table_hbm.at[idx_v.at[t]], slot_dst(h), sems[h])

        def drain(t, h):
            pltpu.make_async_copy(table_hbm.at[idx_v.at[t]], slot_dst(h),
                                  sems[h]).wait()

        def compute(t, h):
            buf = buf0 if h < 2 else buf1
            row0 = (h % 2) * IDX_PER_SLOT

            def reduce_store(r, c, vals):
                while len(vals) > 1:
                    nxt = [a + b for a, b in zip(vals[0::2], vals[1::2])]
                    if len(vals) % 2:
                        nxt.append(vals[-1])
                    vals = nxt
                out_v[t * RS + r, pl.ds(c * L, L)] = vals[0] * (1.0 / S)

            prev = None
            for r in range(RS):
                for c in range(D // L):
                    vals = [buf[row0 + r * S + s, pl.ds(c * L, L)]
                            for s in range(S)]
                    if prev is not None:
                        reduce_store(*prev)
                    prev = (r, c, vals)
            reduce_store(*prev)

        start(0, 0)
        start(1, 1)
        start(2, 2)

        def step(i, carry):
            t4 = i * 4
            for j in range(4):
                t = t4 + j
                drain(t, j)

                @pl.when(t + 3 < NSLOTS)
                def _():
                    start(t + 3, (j + 3) % 4)

                compute(t, j)

            @pl.when(t4 + 4 == NSLOTS // 2)
            def _():
                pltpu.async_copy(out_v.at[pl.ds(0, half)],
                                 out_hbm.at[pl.ds(base_row, half)], osem)

            return carry

        lax.fori_loop(0, NSLOTS // 4, step, 0)
        pltpu.make_async_copy(out_v.at[pl.ds(0, half)],
                              out_hbm.at[pl.ds(base_row, half)], osem).wait()
        pltpu.sync_copy(out_v.at[pl.ds(half, half)],
                        out_hbm.at[pl.ds(base_row + half, half)])

    return sc_mean


_sc_mean = _make_sc_call()


def kernel(nodes, neigh_idx, num_sample, table):
    del nodes, num_sample
    idx = jnp.reshape(neigh_idx.astype(jnp.int32), (NW, NSLOTS, IDX_PER_SLOT))
    return _sc_mean(table, idx)
